# baseline (speedup 1.0000x reference)
import jax
import jax.numpy as jnp
from jax import lax
from jax.experimental import pallas as pl
from jax.experimental.pallas import tpu as pltpu
from jax.experimental.pallas import tpu_sc as plsc

L = 16
CH = 128


def _gmf_body(nc, bpw, d, ut_t, mt_t, uidx_h, midx_h, wb_h, out_h,
              uidx, midx, urT, mrT, wbv, outv, sem):
    wid = lax.axis_index("s") * nc + lax.axis_index("c")
    pltpu.sync_copy(wb_h, wbv)
    pltpu.sync_copy(uidx_h.at[wid], uidx)
    pltpu.sync_copy(midx_h.at[wid], midx)

    def d_body(dd, carry):
        for j in range(bpw // CH):
            sl = pl.ds(pl.multiple_of(j * CH, CH), CH)
            pltpu.async_copy(ut_t.at[dd].at[uidx.at[j]], urT.at[dd, sl], sem)
            pltpu.async_copy(mt_t.at[dd].at[midx.at[j]], mrT.at[dd, sl], sem)
        return carry

    lax.fori_loop(0, d, d_body, 0)
    pltpu.make_async_copy(ut_t.at[pl.ds(0, d), pl.ds(0, bpw)], urT, sem).wait()
    pltpu.make_async_copy(mt_t.at[pl.ds(0, d), pl.ds(0, bpw)], mrT, sem).wait()

    wvecs = [wbv[pl.ds(i * L, L)] for i in range(d // L)]
    ws = [wvecs[k // L][k % L] for k in range(d)]
    bias = wbv[pl.ds(pl.multiple_of(d, L), L)][0]

    def g_body(g, carry):
        sl = pl.ds(pl.multiple_of(g * L, L), L)
        acc = jnp.zeros((L,), jnp.float32)
        for k in range(d):
            acc = acc + urT[k, sl] * mrT[k, sl] * ws[k]
        x = acc + bias
        y = 1.0 / (1.0 + jnp.exp(-x))
        outv[sl] = y
        return carry

    lax.fori_loop(0, bpw // L, g_body, 0)
    pltpu.sync_copy(outv, out_h.at[wid])


def kernel(users, movies, user_table, movie_table, W, b):
    import functools
    batch = users.shape[0]
    d = user_table.shape[1]

    info = plsc.get_sparse_core_info()
    nc, ns = info.num_cores, info.num_subcores
    nw = nc * ns
    bpw = batch // nw

    ut_t = user_table.T
    mt_t = movie_table.T
    users3 = users.astype(jnp.int32).reshape(nw, bpw // CH, CH)
    movies3 = movies.astype(jnp.int32).reshape(nw, bpw // CH, CH)
    wb = jnp.concatenate([W.reshape(-1), b.reshape(-1),
                          jnp.zeros((15,), jnp.float32)])

    mesh = plsc.VectorSubcoreMesh(core_axis_name="c", subcore_axis_name="s")
    run = pl.kernel(
        functools.partial(_gmf_body, nc, bpw, d),
        out_type=jax.ShapeDtypeStruct((nw, bpw), jnp.float32),
        mesh=mesh,
        compiler_params=pltpu.CompilerParams(needs_layout_passes=False,
                                             use_tc_tiling_on_sc=False),
        scratch_types=[
            pltpu.VMEM((bpw // CH, CH), jnp.int32),
            pltpu.VMEM((bpw // CH, CH), jnp.int32),
            pltpu.VMEM((d, bpw), jnp.float32),
            pltpu.VMEM((d, bpw), jnp.float32),
            pltpu.VMEM((d + 16,), jnp.float32),
            pltpu.VMEM((bpw,), jnp.float32),
            pltpu.SemaphoreType.DMA,
        ],
    )
    out = run(ut_t, mt_t, users3, movies3, wb)
    return out.reshape(batch, 1)


# speedup vs baseline: 23.6343x; 23.6343x over previous
import functools

import jax
import jax.numpy as jnp
from jax import lax
from jax.experimental import pallas as pl
from jax.experimental.pallas import tpu as pltpu
from jax.experimental.pallas import tpu_sc as plsc

L = 16
CH = 128
CT = 7813
PLANE = CT * 1024


def _gmf_body(nc, bpw, d, ut_f, mt_f, uidx_h, midx_h, wb_h, out_h,
              uidx, midx, ubase, mbase, pidxu, pidxm, urT, mrT, wbv, outv,
              sem):
    wid = lax.axis_index("s") * nc + lax.axis_index("c")
    pltpu.sync_copy(wb_h, wbv)
    pltpu.sync_copy(uidx_h.at[wid], uidx)
    pltpu.sync_copy(midx_h.at[wid], midx)

    nch = bpw // CH
    ngr = CH // L

    def b_body(t, carry):
        j = t // ngr
        jj = t % ngr
        sl = pl.ds(pl.multiple_of(jj * L, L), L)
        vu = uidx[j, sl]
        vm = midx[j, sl]
        ubase[j, sl] = ((vu >> 7) << 10) | (vu & 127)
        mbase[j, sl] = ((vm >> 7) << 10) | (vm & 127)
        return carry

    lax.fori_loop(0, nch * ngr, b_body, 0)

    def d_body(dd, carry):
        dbase = (dd >> 3) * PLANE + (dd & 7) * CH

        def p_body(t, c2):
            j = t // ngr
            jj = t % ngr
            sl = pl.ds(pl.multiple_of(jj * L, L), L)
            pidxu[dd, j, sl] = ubase[j, sl] + dbase
            pidxm[dd, j, sl] = mbase[j, sl] + dbase
            return c2

        lax.fori_loop(0, nch * ngr, p_body, 0)
        for j in range(nch):
            sl = pl.ds(pl.multiple_of(j * CH, CH), CH)
            pltpu.async_copy(ut_f.at[pidxu.at[dd, j]], urT.at[dd, sl], sem)
            pltpu.async_copy(mt_f.at[pidxm.at[dd, j]], mrT.at[dd, sl], sem)
        return carry

    lax.fori_loop(0, d, d_body, 0)
    pltpu.make_async_copy(out_h, urT, sem).wait()
    pltpu.make_async_copy(out_h, mrT, sem).wait()

    wvecs = [wbv[pl.ds(i * L, L)] for i in range(d // L)]
    ws = [wvecs[k // L][k % L] for k in range(d)]
    bias = wbv[pl.ds(pl.multiple_of(d, L), L)][0]

    def g_body(g, carry):
        sl = pl.ds(pl.multiple_of(g * L, L), L)
        acc = jnp.zeros((L,), jnp.float32)
        for k in range(d):
            acc = acc + urT[k, sl] * mrT[k, sl] * ws[k]
        x = acc + bias
        y = 1.0 / (1.0 + jnp.exp(-x))
        outv[sl] = y
        return carry

    lax.fori_loop(0, bpw // L, g_body, 0)
    pltpu.sync_copy(outv, out_h.at[wid])


def _flat_padded(table):
    p = jnp.pad(table.T, ((0, 0), (0, 64)))
    return (p.reshape(4, 8, CT, 128)
             .transpose(0, 2, 1, 3)
             .reshape(-1))


def kernel(users, movies, user_table, movie_table, W, b):
    batch = users.shape[0]
    d = user_table.shape[1]

    info = plsc.get_sparse_core_info()
    nc, ns = info.num_cores, info.num_subcores
    nw = nc * ns
    bpw = batch // nw

    ut_f = _flat_padded(user_table)
    mt_f = _flat_padded(movie_table)
    users3 = users.astype(jnp.int32).reshape(nw, bpw // CH, CH)
    movies3 = movies.astype(jnp.int32).reshape(nw, bpw // CH, CH)
    wb = jnp.concatenate([W.reshape(-1), b.reshape(-1),
                          jnp.zeros((15,), jnp.float32)])

    mesh = plsc.VectorSubcoreMesh(core_axis_name="c", subcore_axis_name="s")
    run = pl.kernel(
        functools.partial(_gmf_body, nc, bpw, d),
        out_type=jax.ShapeDtypeStruct((nw, bpw), jnp.float32),
        mesh=mesh,
        compiler_params=pltpu.CompilerParams(needs_layout_passes=False,
                                             use_tc_tiling_on_sc=False),
        scratch_types=[
            pltpu.VMEM((bpw // CH, CH), jnp.int32),
            pltpu.VMEM((bpw // CH, CH), jnp.int32),
            pltpu.VMEM((bpw // CH, CH), jnp.int32),
            pltpu.VMEM((bpw // CH, CH), jnp.int32),
            pltpu.VMEM((d, bpw // CH, CH), jnp.int32),
            pltpu.VMEM((d, bpw // CH, CH), jnp.int32),
            pltpu.VMEM((d, bpw), jnp.float32),
            pltpu.VMEM((d, bpw), jnp.float32),
            pltpu.VMEM((d + 16,), jnp.float32),
            pltpu.VMEM((bpw,), jnp.float32),
            pltpu.SemaphoreType.DMA,
        ],
    )
    out = run(ut_f, mt_f, users3, movies3, wb)
    return out.reshape(batch, 1)
